# baseline (speedup 1.0000x reference)
import jax
import jax.numpy as jnp
from jax import lax
from jax.experimental import pallas as pl
from jax.experimental.pallas import tpu as pltpu
from jax.experimental.pallas import tpu_sc as plsc

_INFO = plsc.get_sparse_core_info()
_NC = _INFO.num_cores
_NS = _INFO.num_subcores
_NW = _NC * _NS

_B = 16384 * 50
_D = 64
_DP = 128
_G = 200
_NBUF = 4
_LOOK = _NBUF - 1
_PER_W = _B // _NW
_NG = _PER_W // _G


def _body(tok_hbm, w_hbm, out_hbm, idx_v, b0, b1, b2, b3,
          g0s, g1s, g2s, g3s, w0s, w1s, w2s, w3s):
    bufs = (b0, b1, b2, b3)
    gsems = (g0s, g1s, g2s, g3s)
    wsems = (w0s, w1s, w2s, w3s)

    wid = lax.axis_index("s") * _NC + lax.axis_index("c")
    base = wid * _PER_W
    pltpu.sync_copy(tok_hbm.at[pl.ds(base, _PER_W)], idx_v)

    def fire(g, b):
        pltpu.async_copy(
            w_hbm.at[idx_v.at[pl.ds(g * _G, _G)]], bufs[b], gsems[b])

    for g in range(_LOOK):
        fire(g, g)

    def outer(t, carry):
        for b in range(_NBUF):
            s = t * _NBUF + b
            nb = (b + _LOOK) % _NBUF

            @pl.when(s + _LOOK < _NG)
            def _():
                @pl.when(s + _LOOK >= _NBUF)
                def _():
                    pltpu.make_async_copy(
                        bufs[nb], out_hbm.at[pl.ds(0, _G)], wsems[nb]).wait()
                fire(s + _LOOK, nb)

            pltpu.make_async_copy(
                w_hbm.at[idx_v.at[pl.ds(s * _G, _G)]], bufs[b], gsems[b]).wait()
            pltpu.async_copy(
                bufs[b], out_hbm.at[pl.ds(base + s * _G, _G)], wsems[b])
        return carry

    lax.fori_loop(0, _NG // _NBUF, outer, 0)

    for b in range(_NBUF):
        pltpu.make_async_copy(
            bufs[b], out_hbm.at[pl.ds(0, _G)], wsems[b]).wait()


def kernel(token_ids, weight):
    tok = token_ids.reshape(_B)
    wpad = jnp.pad(weight, ((0, 0), (0, _DP - _D)))
    mesh = plsc.VectorSubcoreMesh(core_axis_name="c", subcore_axis_name="s")
    out = pl.kernel(
        _body,
        mesh=mesh,
        compiler_params=pltpu.CompilerParams(use_tc_tiling_on_sc=True),
        out_type=jax.ShapeDtypeStruct((_B, _DP), jnp.float32),
        scratch_types=[
            pltpu.VMEM((_PER_W,), jnp.int32)]
        + [pltpu.VMEM((_G, _DP), jnp.float32) for _ in range(_NBUF)]
        + [pltpu.SemaphoreType.DMA for _ in range(2 * _NBUF)],
    )(tok, wpad)
    return out[:, :_D].reshape(token_ids.shape + (_D,))
